# baseline (speedup 1.0000x reference)
import functools

import jax
import jax.numpy as jnp
from jax import lax
from jax.experimental import pallas as pl
from jax.experimental.pallas import tpu as pltpu
from jax.experimental.pallas import tpu_sc as plsc

BATCH = 16384
DIM = 32
ROWS = 1000000
PACK = 4
PROWS = ROWS // PACK
PADW = 128
NC = 2
NS = 16
NW = NC * NS
B_PER_W = BATCH // NW
CHUNK = 128
NCHUNK = B_PER_W // CHUNK
LANES = 16
WEIGHT_DECAY = 0.025


def _sc_gather(u2d, i2d, j2d, Wp, Hp):
    mesh = plsc.VectorSubcoreMesh(core_axis_name="c", subcore_axis_name="s")
    out = jax.ShapeDtypeStruct((BATCH, PADW), jnp.float32)

    @functools.partial(
        pl.kernel,
        mesh=mesh,
        out_type=(out, out, out),
        compiler_params=pltpu.CompilerParams(use_tc_tiling_on_sc=False),
        scratch_types=[
            pltpu.VMEM((NCHUNK, CHUNK), jnp.int32),
            pltpu.VMEM((B_PER_W, PADW), jnp.float32),
            pltpu.SemaphoreType.DMA,
        ],
    )
    def k(u_hbm, i_hbm, j_hbm, w_hbm, h_hbm, ou_hbm, oi_hbm, oj_hbm,
          ix, rows, sem):
        wid = lax.axis_index("s") * NC + lax.axis_index("c")
        base = wid * B_PER_W
        row0 = wid * NCHUNK

        for idx_hbm, tab_hbm, o_hbm in (
            (u_hbm, w_hbm, ou_hbm),
            (i_hbm, h_hbm, oi_hbm),
            (j_hbm, h_hbm, oj_hbm),
        ):
            pltpu.sync_copy(idx_hbm.at[pl.ds(row0, NCHUNK)], ix)
            for c in range(NCHUNK):
                for l in range(CHUNK // LANES):
                    s = pl.ds(l * LANES, LANES)
                    ix[c, s] = lax.shift_right_logical(ix[c, s], 2)
            copies = []
            for c in range(NCHUNK):
                copies.append(pltpu.async_copy(
                    tab_hbm.at[ix.at[c]], rows.at[pl.ds(c * CHUNK, CHUNK)],
                    sem))
            for cp in copies:
                cp.wait()
            pltpu.sync_copy(rows, o_hbm.at[pl.ds(base, B_PER_W)])

    return k(u2d, i2d, j2d, Wp, Hp)


PAD_BLK = 2048


def _tc_pack_body(wt_ref, out_ref):
    x = wt_ref[...]
    x3 = x.reshape(DIM, PAD_BLK // PACK, PACK)
    eye = jnp.eye(DIM, dtype=jnp.float32)
    cols = []
    for q in range(PACK):
        xq = x3[:, :, q]
        cols.append(jax.lax.dot_general(
            xq, eye, (((0,), (0,)), ((), ())),
            preferred_element_type=jnp.float32))
    out_ref[...] = jnp.concatenate(cols, axis=1)


def _tc_pack(Wt):
    return pl.pallas_call(
        _tc_pack_body,
        grid=(pl.cdiv(ROWS, PAD_BLK),),
        in_specs=(pl.BlockSpec((DIM, PAD_BLK), lambda c: (0, c)),),
        out_shape=jax.ShapeDtypeStruct((PROWS, PADW), jnp.float32),
        out_specs=pl.BlockSpec((PAD_BLK // PACK, PADW), lambda c: (c, 0)),
        compiler_params=pltpu.CompilerParams(
            dimension_semantics=("parallel",)),
    )(Wt)


TC_GRID = 8
TB = BATCH // TC_GRID


def _pick(g, q):
    acc = jnp.zeros((TB, DIM), jnp.float32)
    qq = (q & 3).reshape(TB, 1)
    for w in range(PACK):
        acc = acc + jnp.where(qq == w, g[:, w * DIM:(w + 1) * DIM], 0.0)
    return acc


def _tc_loss_body(gu_ref, gi_ref, gj_ref, u_ref, i_ref, j_ref,
                  loss_ref, reg_ref):
    step = pl.program_id(0)
    u = _pick(gu_ref[...], u_ref[...].reshape(TB))
    hi = _pick(gi_ref[...], i_ref[...].reshape(TB))
    hj = _pick(gj_ref[...], j_ref[...].reshape(TB))
    x_ui = jnp.sum(u * hi, axis=1)
    x_uj = jnp.sum(u * hj, axis=1)
    x_uij = jnp.clip(x_ui - x_uj, -80.0, 100000000.0)
    z = -x_uij
    softplus = jnp.maximum(z, 0.0) + jnp.log1p(jnp.exp(-jnp.abs(z)))
    reg = WEIGHT_DECAY * (jnp.sum(u * u) + jnp.sum(hi * hi) + jnp.sum(hj * hj))
    part = jnp.sum(softplus) + reg

    @pl.when(step == 0)
    def _():
        loss_ref[0, 0] = part
        reg_ref[0, 0] = reg

    @pl.when(step != 0)
    def _():
        loss_ref[0, 0] += part
        reg_ref[0, 0] += reg


def _tc_loss(gu, gi, gj, u2d, i2d, j2d):
    scalar = jax.ShapeDtypeStruct((1, 1), jnp.float32)
    g_spec = pl.BlockSpec((TB, PADW), lambda s: (s, 0))
    q_spec = pl.BlockSpec((TB // CHUNK, CHUNK), lambda s: (s, 0))
    return pl.pallas_call(
        _tc_loss_body,
        grid=(TC_GRID,),
        in_specs=(g_spec, g_spec, g_spec, q_spec, q_spec, q_spec),
        out_shape=(scalar, scalar),
        out_specs=(pl.BlockSpec(memory_space=pltpu.SMEM),
                   pl.BlockSpec(memory_space=pltpu.SMEM)),
    )(gu, gi, gj, u2d, i2d, j2d)


def kernel(u, i, j, adv, W, H):
    shape2d = (BATCH // CHUNK, CHUNK)
    u2d = u.reshape(shape2d)
    i2d = i.reshape(shape2d)
    j2d = j.reshape(shape2d)
    Wp = _tc_pack(W.T)
    Hp = _tc_pack(H.T)
    gu, gi, gj = _sc_gather(u2d, i2d, j2d, Wp, Hp)
    loss, reg = _tc_loss(gu, gi, gj, u2d, i2d, j2d)
    total = loss[0, 0]
    if adv is True:
        total = total + reg[0, 0]
    return total


# speedup vs baseline: 21.4404x; 21.4404x over previous
import functools

import jax
import jax.numpy as jnp
from jax import lax
from jax.experimental import pallas as pl
from jax.experimental.pallas import tpu as pltpu
from jax.experimental.pallas import tpu_sc as plsc

BATCH = 16384
DIM = 32
ROWS = 1000000
PADW = 128
NC = 2
NS = 16
NW = NC * NS
B_PER_W = BATCH // NW
CHUNK = 128
NCHUNK = B_PER_W // CHUNK
WEIGHT_DECAY = 0.025


def _sc_gather(u2d, i2d, j2d, Wb, Hb):
    mesh = plsc.VectorSubcoreMesh(core_axis_name="c", subcore_axis_name="s")
    out = jax.ShapeDtypeStruct((BATCH, PADW), jnp.float32)

    @functools.partial(
        pl.kernel,
        mesh=mesh,
        out_type=(out, out, out),
        compiler_params=pltpu.CompilerParams(use_tc_tiling_on_sc=False),
        scratch_types=[
            pltpu.VMEM((NCHUNK, CHUNK), jnp.int32),
            pltpu.VMEM((B_PER_W, PADW), jnp.float32),
            pltpu.SemaphoreType.DMA,
        ],
    )
    def k(u_hbm, i_hbm, j_hbm, w_hbm, h_hbm, ou_hbm, oi_hbm, oj_hbm,
          ix, rows, sem):
        wid = lax.axis_index("s") * NC + lax.axis_index("c")
        base = wid * B_PER_W
        row0 = wid * NCHUNK

        for idx_hbm, tab_hbm, o_hbm in (
            (u_hbm, w_hbm, ou_hbm),
            (i_hbm, h_hbm, oi_hbm),
            (j_hbm, h_hbm, oj_hbm),
        ):
            pltpu.sync_copy(idx_hbm.at[pl.ds(row0, NCHUNK)], ix)
            copies = []
            for c in range(NCHUNK):
                copies.append(pltpu.async_copy(
                    tab_hbm.at[ix.at[c]], rows.at[pl.ds(c * CHUNK, CHUNK)],
                    sem))
            for cp in copies:
                cp.wait()
            pltpu.sync_copy(rows, o_hbm.at[pl.ds(base, B_PER_W)])

    return k(u2d, i2d, j2d, Wb, Hb)


TC_GRID = 8
TB = BATCH // TC_GRID


def _tc_loss_body(gu_ref, gi_ref, gj_ref, loss_ref, reg_ref):
    step = pl.program_id(0)
    u = gu_ref[:, :DIM]
    hi = gi_ref[:, :DIM]
    hj = gj_ref[:, :DIM]
    x_ui = jnp.sum(u * hi, axis=1)
    x_uj = jnp.sum(u * hj, axis=1)
    x_uij = jnp.clip(x_ui - x_uj, -80.0, 100000000.0)
    z = -x_uij
    softplus = jnp.maximum(z, 0.0) + jnp.log1p(jnp.exp(-jnp.abs(z)))
    reg = WEIGHT_DECAY * (jnp.sum(u * u) + jnp.sum(hi * hi) + jnp.sum(hj * hj))
    part = jnp.sum(softplus) + reg

    @pl.when(step == 0)
    def _():
        loss_ref[0, 0] = part
        reg_ref[0, 0] = reg

    @pl.when(step != 0)
    def _():
        loss_ref[0, 0] += part
        reg_ref[0, 0] += reg


def _tc_loss(gu, gi, gj):
    scalar = jax.ShapeDtypeStruct((1, 1), jnp.float32)
    g_spec = pl.BlockSpec((TB, PADW), lambda s: (s, 0))
    return pl.pallas_call(
        _tc_loss_body,
        grid=(TC_GRID,),
        in_specs=(g_spec, g_spec, g_spec),
        out_shape=(scalar, scalar),
        out_specs=(pl.BlockSpec(memory_space=pltpu.SMEM),
                   pl.BlockSpec(memory_space=pltpu.SMEM)),
    )(gu, gi, gj)


PAD_BLK = 8192


def _tc_pad_body(wt_ref, out_ref):
    x = wt_ref[...]
    eye = jnp.eye(DIM, dtype=jnp.float32)
    xt = jax.lax.dot_general(
        x, eye, (((0,), (0,)), ((), ())),
        preferred_element_type=jnp.float32)
    out_ref[:, :DIM] = xt


def _tc_pad(Wt):
    return pl.pallas_call(
        _tc_pad_body,
        grid=(pl.cdiv(ROWS, PAD_BLK),),
        in_specs=(pl.BlockSpec((DIM, PAD_BLK), lambda c: (0, c)),),
        out_shape=jax.ShapeDtypeStruct((ROWS, PADW), jnp.float32),
        out_specs=pl.BlockSpec((PAD_BLK, PADW), lambda c: (c, 0)),
        compiler_params=pltpu.CompilerParams(
            dimension_semantics=("parallel",)),
    )(Wt)


def kernel(u, i, j, adv, W, H):
    shape2d = (BATCH // CHUNK, CHUNK)
    Wb = _tc_pad(W.T)
    Hb = _tc_pad(H.T)
    gu, gi, gj = _sc_gather(u.reshape(shape2d), i.reshape(shape2d),
                            j.reshape(shape2d), Wb, Hb)
    loss, reg = _tc_loss(gu, gi, gj)
    total = loss[0, 0]
    if adv is True:
        total = total + reg[0, 0]
    return total


# speedup vs baseline: 24.8504x; 1.1590x over previous
import functools

import jax
import jax.numpy as jnp
from jax import lax
from jax.experimental import pallas as pl
from jax.experimental.pallas import tpu as pltpu
from jax.experimental.pallas import tpu_sc as plsc

BATCH = 16384
DIM = 32
ROWS = 1000000
PADW = 128
NC = 2
NS = 16
NW = NC * NS
B_PER_W = BATCH // NW
CHUNK = 128
NCHUNK = B_PER_W // CHUNK
WEIGHT_DECAY = 0.025


def _sc_gather(u2d, i2d, j2d, Wb, Hb):
    mesh = plsc.VectorSubcoreMesh(core_axis_name="c", subcore_axis_name="s")
    out = jax.ShapeDtypeStruct((BATCH, PADW), jnp.float32)

    @functools.partial(
        pl.kernel,
        mesh=mesh,
        out_type=(out, out, out),
        compiler_params=pltpu.CompilerParams(use_tc_tiling_on_sc=False),
        scratch_types=[
            pltpu.VMEM((NCHUNK, CHUNK), jnp.int32),
            pltpu.VMEM((B_PER_W, PADW), jnp.float32),
            pltpu.SemaphoreType.DMA,
        ],
    )
    def k(u_hbm, i_hbm, j_hbm, w_hbm, h_hbm, ou_hbm, oi_hbm, oj_hbm,
          ix, rows, sem):
        wid = lax.axis_index("s") * NC + lax.axis_index("c")
        base = wid * B_PER_W
        row0 = wid * NCHUNK

        for idx_hbm, tab_hbm, o_hbm in (
            (u_hbm, w_hbm, ou_hbm),
            (i_hbm, h_hbm, oi_hbm),
            (j_hbm, h_hbm, oj_hbm),
        ):
            pltpu.sync_copy(idx_hbm.at[pl.ds(row0, NCHUNK)], ix)
            copies = []
            for c in range(NCHUNK):
                copies.append(pltpu.async_copy(
                    tab_hbm.at[ix.at[c]], rows.at[pl.ds(c * CHUNK, CHUNK)],
                    sem))
            for cp in copies:
                cp.wait()
            pltpu.sync_copy(rows, o_hbm.at[pl.ds(base, B_PER_W)])

    return k(u2d, i2d, j2d, Wb, Hb)


TC_GRID = 8
TB = BATCH // TC_GRID


def _tc_loss_body(gu_ref, gi_ref, gj_ref, loss_ref, reg_ref):
    step = pl.program_id(0)
    u = gu_ref[:, :DIM]
    hi = gi_ref[:, :DIM]
    hj = gj_ref[:, :DIM]
    x_ui = jnp.sum(u * hi, axis=1)
    x_uj = jnp.sum(u * hj, axis=1)
    x_uij = jnp.clip(x_ui - x_uj, -80.0, 100000000.0)
    z = -x_uij
    softplus = jnp.maximum(z, 0.0) + jnp.log1p(jnp.exp(-jnp.abs(z)))
    reg = WEIGHT_DECAY * (jnp.sum(u * u) + jnp.sum(hi * hi) + jnp.sum(hj * hj))
    part = jnp.sum(softplus) + reg

    @pl.when(step == 0)
    def _():
        loss_ref[0, 0] = part
        reg_ref[0, 0] = reg

    @pl.when(step != 0)
    def _():
        loss_ref[0, 0] += part
        reg_ref[0, 0] += reg


def _tc_loss(gu, gi, gj):
    scalar = jax.ShapeDtypeStruct((1, 1), jnp.float32)
    g_spec = pl.BlockSpec((TB, PADW), lambda s: (s, 0))
    return pl.pallas_call(
        _tc_loss_body,
        grid=(TC_GRID,),
        in_specs=(g_spec, g_spec, g_spec),
        out_shape=(scalar, scalar),
        out_specs=(pl.BlockSpec(memory_space=pltpu.SMEM),
                   pl.BlockSpec(memory_space=pltpu.SMEM)),
    )(gu, gi, gj)


PAD_BLK = 16384


def _tc_pad_body(wt_ref, out_ref):
    x = wt_ref[...]
    eye = jnp.eye(DIM, dtype=jnp.float32)
    xt = jax.lax.dot_general(
        x, eye, (((0,), (0,)), ((), ())),
        preferred_element_type=jnp.float32)
    out_ref[:, :DIM] = xt


def _tc_pad(Wt):
    return pl.pallas_call(
        _tc_pad_body,
        grid=(pl.cdiv(ROWS, PAD_BLK),),
        in_specs=(pl.BlockSpec((DIM, PAD_BLK), lambda c: (0, c)),),
        out_shape=jax.ShapeDtypeStruct((ROWS, PADW), jnp.float32),
        out_specs=pl.BlockSpec((PAD_BLK, PADW), lambda c: (c, 0)),
        compiler_params=pltpu.CompilerParams(
            dimension_semantics=("parallel",)),
    )(Wt)


def kernel(u, i, j, adv, W, H):
    shape2d = (BATCH // CHUNK, CHUNK)
    Wb = _tc_pad(W.T)
    Hb = _tc_pad(H.T)
    gu, gi, gj = _sc_gather(u.reshape(shape2d), i.reshape(shape2d),
                            j.reshape(shape2d), Wb, Hb)
    loss, reg = _tc_loss(gu, gi, gj)
    total = loss[0, 0]
    if adv is True:
        total = total + reg[0, 0]
    return total


# speedup vs baseline: 25.5319x; 1.0274x over previous
import functools

import jax
import jax.numpy as jnp
from jax import lax
from jax.experimental import pallas as pl
from jax.experimental.pallas import tpu as pltpu
from jax.experimental.pallas import tpu_sc as plsc

BATCH = 16384
DIM = 32
ROWS = 1000000
PADW = 128
NC = 2
NS = 16
NW = NC * NS
B_PER_W = BATCH // NW
CHUNK = 128
NCHUNK = B_PER_W // CHUNK
WEIGHT_DECAY = 0.025


def _sc_gather(u2d, i2d, j2d, Wb, Hb):
    mesh = plsc.VectorSubcoreMesh(core_axis_name="c", subcore_axis_name="s")
    out = jax.ShapeDtypeStruct((BATCH, PADW), jnp.float32)

    @functools.partial(
        pl.kernel,
        mesh=mesh,
        out_type=(out, out, out),
        compiler_params=pltpu.CompilerParams(use_tc_tiling_on_sc=False),
        scratch_types=[
            pltpu.VMEM((NCHUNK, CHUNK), jnp.int32),
            pltpu.VMEM((B_PER_W, PADW), jnp.float32),
            pltpu.SemaphoreType.DMA,
        ],
    )
    def k(u_hbm, i_hbm, j_hbm, w_hbm, h_hbm, ou_hbm, oi_hbm, oj_hbm,
          ix, rows, sem):
        wid = lax.axis_index("s") * NC + lax.axis_index("c")
        base = wid * B_PER_W
        row0 = wid * NCHUNK

        for idx_hbm, tab_hbm, o_hbm in (
            (u_hbm, w_hbm, ou_hbm),
            (i_hbm, h_hbm, oi_hbm),
            (j_hbm, h_hbm, oj_hbm),
        ):
            pltpu.sync_copy(idx_hbm.at[pl.ds(row0, NCHUNK)], ix)
            copies = []
            for c in range(NCHUNK):
                copies.append(pltpu.async_copy(
                    tab_hbm.at[ix.at[c]], rows.at[pl.ds(c * CHUNK, CHUNK)],
                    sem))
            for cp in copies:
                cp.wait()
            pltpu.sync_copy(rows, o_hbm.at[pl.ds(base, B_PER_W)])

    return k(u2d, i2d, j2d, Wb, Hb)


TC_GRID = 8
TB = BATCH // TC_GRID


def _tc_loss_body(gu_ref, gi_ref, gj_ref, loss_ref, reg_ref):
    step = pl.program_id(0)
    u = gu_ref[:, :DIM]
    hi = gi_ref[:, :DIM]
    hj = gj_ref[:, :DIM]
    x_ui = jnp.sum(u * hi, axis=1)
    x_uj = jnp.sum(u * hj, axis=1)
    x_uij = jnp.clip(x_ui - x_uj, -80.0, 100000000.0)
    z = -x_uij
    softplus = jnp.maximum(z, 0.0) + jnp.log1p(jnp.exp(-jnp.abs(z)))
    reg = WEIGHT_DECAY * (jnp.sum(u * u) + jnp.sum(hi * hi) + jnp.sum(hj * hj))
    part = jnp.sum(softplus) + reg

    @pl.when(step == 0)
    def _():
        loss_ref[0, 0] = part
        reg_ref[0, 0] = reg

    @pl.when(step != 0)
    def _():
        loss_ref[0, 0] += part
        reg_ref[0, 0] += reg


def _tc_loss(gu, gi, gj):
    scalar = jax.ShapeDtypeStruct((1, 1), jnp.float32)
    g_spec = pl.BlockSpec((TB, PADW), lambda s: (s, 0))
    return pl.pallas_call(
        _tc_loss_body,
        grid=(TC_GRID,),
        in_specs=(g_spec, g_spec, g_spec),
        out_shape=(scalar, scalar),
        out_specs=(pl.BlockSpec(memory_space=pltpu.SMEM),
                   pl.BlockSpec(memory_space=pltpu.SMEM)),
    )(gu, gi, gj)


PAD_BLK = 32768


def _tc_pad_body(wt_ref, out_ref):
    x = wt_ref[...]
    eye = jnp.eye(DIM, dtype=jnp.float32)
    xt = jax.lax.dot_general(
        x, eye, (((0,), (0,)), ((), ())),
        preferred_element_type=jnp.float32)
    out_ref[:, :DIM] = xt


def _tc_pad(Wt):
    return pl.pallas_call(
        _tc_pad_body,
        grid=(pl.cdiv(ROWS, PAD_BLK),),
        in_specs=(pl.BlockSpec((DIM, PAD_BLK), lambda c: (0, c)),),
        out_shape=jax.ShapeDtypeStruct((ROWS, PADW), jnp.float32),
        out_specs=pl.BlockSpec((PAD_BLK, PADW), lambda c: (c, 0)),
        compiler_params=pltpu.CompilerParams(
            dimension_semantics=("parallel",)),
    )(Wt)


def kernel(u, i, j, adv, W, H):
    shape2d = (BATCH // CHUNK, CHUNK)
    Wb = _tc_pad(W.T)
    Hb = _tc_pad(H.T)
    gu, gi, gj = _sc_gather(u.reshape(shape2d), i.reshape(shape2d),
                            j.reshape(shape2d), Wb, Hb)
    loss, reg = _tc_loss(gu, gi, gj)
    total = loss[0, 0]
    if adv is True:
        total = total + reg[0, 0]
    return total
